# Initial kernel scaffold; baseline (speedup 1.0000x reference)
import jax
import jax.numpy as jnp
from jax.experimental import pallas as pl


def kernel(r, xyz, a, N, embed, W_in, Wf1, bf1, Wf2, bf2, Wo1, bo1, Wo2, bo2, Wa1, ba1, Wa2, ba2):
    raise NotImplementedError("write your pallas kernel here")


# speedup vs baseline: 1.7206x; 1.7206x over previous
import functools

import jax
import jax.numpy as jnp
from jax import lax
from jax.experimental import pallas as pl
from jax.experimental.pallas import tpu as pltpu
from jax.experimental.pallas import tpu_sc as plsc

N_ATOM_BASIS = 128
N_FILTERS = 128
N_GAUSS = 25
CUTOFF = 5.0
T = 3
NA = 10000
E = 320000

NC = 2
NS = 16
NW = NC * NS
EPT = E // NW
LOG2 = 0.6931471805599453
GWIDTH = CUTOFF / (N_GAUSS - 1)

_sc_mesh = functools.partial(
    plsc.VectorSubcoreMesh, core_axis_name="c", subcore_axis_name="s",
    num_cores=NC, num_subcores=NS)


def _ssp(x):
    return jnp.maximum(x, 0.0) + jnp.log(1.0 + jnp.exp(-jnp.abs(x))) - LOG2


_DCH = 2000


def _sc_dist_body(x_hbm, y_hbm, z_hbm, src_hbm, dst_hbm, s_out,
                  xv, yv, zv, isrc, idst, sv):
    c = lax.axis_index("c")
    s = lax.axis_index("s")
    base = (c * NS + s) * EPT
    pltpu.sync_copy(x_hbm, xv)
    pltpu.sync_copy(y_hbm, yv)
    pltpu.sync_copy(z_hbm, zv)

    def chunk(ci, carry):
        off = base + ci * _DCH
        pltpu.sync_copy(src_hbm.at[pl.ds(off, _DCH)], isrc)
        pltpu.sync_copy(dst_hbm.at[pl.ds(off, _DCH)], idst)

        def grp(g, carry2):
            sl = pl.ds(g * 16, 16)
            i_d = idst[sl]
            i_s = isrc[sl]
            dx = plsc.load_gather(xv, [i_d]) - plsc.load_gather(xv, [i_s])
            dy = plsc.load_gather(yv, [i_d]) - plsc.load_gather(yv, [i_s])
            dz = plsc.load_gather(zv, [i_d]) - plsc.load_gather(zv, [i_s])
            sv[sl] = dx * dx + dy * dy + dz * dz
            return carry2

        lax.fori_loop(0, _DCH // 16, grp, 0)
        pltpu.sync_copy(sv, s_out.at[pl.ds(off, _DCH)])
        return carry

    lax.fori_loop(0, EPT // _DCH, chunk, 0)


def _sc_dist(x, y, z, src, dst):
    return pl.kernel(
        _sc_dist_body,
        out_type=jax.ShapeDtypeStruct((E,), jnp.float32),
        mesh=_sc_mesh(),
        scratch_types=[
            pltpu.VMEM((NA,), jnp.float32),
            pltpu.VMEM((NA,), jnp.float32),
            pltpu.VMEM((NA,), jnp.float32),
            pltpu.VMEM((_DCH,), jnp.int32),
            pltpu.VMEM((_DCH,), jnp.int32),
            pltpu.VMEM((_DCH,), jnp.float32),
        ],
        compiler_params=pltpu.CompilerParams(needs_layout_passes=False),
    )(x, y, z, src, dst)


_MCH = 80
_NMCH = EPT // _MCH


def _sc_msg_body(t, xn_hbm, w_hbm, src_hbm, dst_hbm, zeros_hbm, out_hbm,
                 isrc, idst, wv, xv, aggs, sem):
    c = lax.axis_index("c")
    s = lax.axis_index("s")
    base = (c * NS + s) * EPT

    @pl.when(s == 0)
    def _zero():
        pltpu.sync_copy(zeros_hbm, aggs)

    plsc.subcore_barrier()

    def chunk(ci, carry):
        off = base + ci * _MCH
        pltpu.sync_copy(src_hbm.at[pl.ds(off, _MCH)], isrc)
        pltpu.sync_copy(dst_hbm.at[pl.ds(off, _MCH)], idst)
        cp = pltpu.async_copy(xn_hbm.at[isrc], xv, sem)
        pltpu.sync_copy(w_hbm.at[t, pl.ds(off, _MCH), :], wv)
        cp.wait()

        def mul(j, carry2):
            for k in range(8):
                sl = pl.ds(k * 16, 16)
                wv[j, sl] = wv[j, sl] * xv[j, sl]
            return carry2

        lax.fori_loop(0, _MCH, mul, 0)
        pltpu.sync_copy(wv, aggs.at[idst], add=True)
        return carry

    lax.fori_loop(0, _NMCH, chunk, 0)
    plsc.subcore_barrier()

    @pl.when(s < NS - 1)
    def _out_main():
        pltpu.sync_copy(aggs.at[pl.ds(s * 640, 640), :],
                        out_hbm.at[c, pl.ds(s * 640, 640), :])

    @pl.when(s == NS - 1)
    def _out_tail():
        pltpu.sync_copy(aggs.at[pl.ds(9600, 400), :],
                        out_hbm.at[c, pl.ds(9600, 400), :])


def _sc_msg(t, xn, w_all, src, dst, zeros):
    return pl.kernel(
        functools.partial(_sc_msg_body, t),
        out_type=jax.ShapeDtypeStruct((NC, NA, N_FILTERS), jnp.float32),
        mesh=_sc_mesh(),
        scratch_types=[
            pltpu.VMEM((_MCH,), jnp.int32),
            pltpu.VMEM((_MCH,), jnp.int32),
            pltpu.VMEM((_MCH, N_FILTERS), jnp.float32),
            pltpu.VMEM((_MCH, N_FILTERS), jnp.float32),
            pltpu.VMEM_SHARED((NA, N_FILTERS), jnp.float32),
            pltpu.SemaphoreType.DMA,
        ],
    )(xn, w_all, src, dst, zeros)


_BN = 1000


def _tc_embed_body(r_ref, emb_ref, win_ref, h_ref, xn_ref):
    rv = r_ref[...]
    ids = lax.broadcasted_iota(jnp.int32, (_BN, 128), 1)
    onehot = (ids == rv).astype(jnp.float32)
    h = jnp.dot(onehot, emb_ref[...], preferred_element_type=jnp.float32)
    h_ref[...] = h
    xn_ref[...] = jnp.dot(h, win_ref[...], preferred_element_type=jnp.float32)


def _tc_embed(r, emb_pad, w_in0):
    return pl.pallas_call(
        _tc_embed_body,
        grid=(NA // _BN,),
        in_specs=[
            pl.BlockSpec((_BN, 1), lambda i: (i, 0)),
            pl.BlockSpec((128, 128), lambda i: (0, 0)),
            pl.BlockSpec((128, 128), lambda i: (0, 0)),
        ],
        out_specs=[
            pl.BlockSpec((_BN, 128), lambda i: (i, 0)),
            pl.BlockSpec((_BN, 128), lambda i: (i, 0)),
        ],
        out_shape=[
            jax.ShapeDtypeStruct((NA, 128), jnp.float32),
            jax.ShapeDtypeStruct((NA, 128), jnp.float32),
        ],
    )(r, emb_pad, w_in0)


_BE = 1000


def _tc_filter_body(s_ref, wf1_ref, bf1_ref, wf2_ref, bf2_ref, out_ref):
    e = jnp.sqrt(s_ref[...])
    off = lax.broadcasted_iota(jnp.int32, (1, 128), 1).astype(jnp.float32) * GWIDTH
    d = e - off
    g = jnp.exp((-0.5 / (GWIDTH * GWIDTH)) * d * d)
    w1 = _ssp(jnp.dot(g, wf1_ref[0], preferred_element_type=jnp.float32)
              + bf1_ref[0])
    w2 = jnp.dot(w1, wf2_ref[0], preferred_element_type=jnp.float32) + bf2_ref[0]
    fcut = 0.5 * (jnp.cos((jnp.pi / CUTOFF) * e) + 1.0)
    fcut = fcut * (e < CUTOFF).astype(jnp.float32)
    out_ref[0] = w2 * fcut


def _tc_filter(s2, wf1p, bf1, wf2, bf2):
    return pl.pallas_call(
        _tc_filter_body,
        grid=(T, E // _BE),
        in_specs=[
            pl.BlockSpec((_BE, 1), lambda t, i: (i, 0)),
            pl.BlockSpec((1, 128, 128), lambda t, i: (t, 0, 0)),
            pl.BlockSpec((1, 1, 128), lambda t, i: (t, 0, 0)),
            pl.BlockSpec((1, 128, 128), lambda t, i: (t, 0, 0)),
            pl.BlockSpec((1, 1, 128), lambda t, i: (t, 0, 0)),
        ],
        out_specs=pl.BlockSpec((1, _BE, 128), lambda t, i: (t, i, 0)),
        out_shape=jax.ShapeDtypeStruct((T, E, 128), jnp.float32),
    )(s2, wf1p, bf1, wf2, bf2)


def _tc_update_body(p0_ref, p1_ref, h_ref, wo1_ref, bo1_ref, wo2_ref, bo2_ref,
                    win_ref, hn_ref, xn_ref):
    agg = p0_ref[0] + p1_ref[0]
    t1 = _ssp(jnp.dot(agg, wo1_ref[...], preferred_element_type=jnp.float32)
              + bo1_ref[...])
    dr = jnp.dot(t1, wo2_ref[...], preferred_element_type=jnp.float32) + bo2_ref[...]
    hn = h_ref[...] + dr
    hn_ref[...] = hn
    xn_ref[...] = jnp.dot(hn, win_ref[...], preferred_element_type=jnp.float32)


def _tc_update(parts, h, wo1, bo1, wo2, bo2, w_in_next):
    return pl.pallas_call(
        _tc_update_body,
        grid=(NA // _BN,),
        in_specs=[
            pl.BlockSpec((1, _BN, 128), lambda i: (0, i, 0)),
            pl.BlockSpec((1, _BN, 128), lambda i: (1, i, 0)),
            pl.BlockSpec((_BN, 128), lambda i: (i, 0)),
            pl.BlockSpec((128, 128), lambda i: (0, 0)),
            pl.BlockSpec((1, 128), lambda i: (0, 0)),
            pl.BlockSpec((128, 128), lambda i: (0, 0)),
            pl.BlockSpec((1, 128), lambda i: (0, 0)),
            pl.BlockSpec((128, 128), lambda i: (0, 0)),
        ],
        out_specs=[
            pl.BlockSpec((_BN, 128), lambda i: (i, 0)),
            pl.BlockSpec((_BN, 128), lambda i: (i, 0)),
        ],
        out_shape=[
            jax.ShapeDtypeStruct((NA, 128), jnp.float32),
            jax.ShapeDtypeStruct((NA, 128), jnp.float32),
        ],
    )(parts, parts, h, wo1, bo1, wo2, bo2, w_in_next)


def _tc_head_body(h_ref, wa1_ref, ba1_ref, wa2_ref, ba2_ref, cn_ref, out_ref):
    i = pl.program_id(0)

    @pl.when(i == 0)
    def _init():
        out_ref[...] = jnp.zeros_like(out_ref)

    t1 = _ssp(jnp.dot(h_ref[...], wa1_ref[...],
                      preferred_element_type=jnp.float32) + ba1_ref[...])
    y = jnp.dot(t1, wa2_ref[...], preferred_element_type=jnp.float32)
    rows = i * _BN + lax.broadcasted_iota(jnp.int32, (_BN, 1), 0)
    m0 = (rows < cn_ref[0, 0]).astype(jnp.float32)
    s_all = jnp.sum(y)
    s0 = jnp.sum(y * m0)
    upd = jnp.concatenate(
        [jnp.full((1, 1), s0, jnp.float32),
         jnp.full((1, 1), s_all - s0, jnp.float32)], axis=0)
    out_ref[...] = out_ref[...] + upd


def _tc_head(h, wa1, ba1, wa2, ba2, cn):
    return pl.pallas_call(
        _tc_head_body,
        grid=(NA // _BN,),
        in_specs=[
            pl.BlockSpec((_BN, 128), lambda i: (i, 0)),
            pl.BlockSpec((128, 64), lambda i: (0, 0)),
            pl.BlockSpec((1, 64), lambda i: (0, 0)),
            pl.BlockSpec((64, 1), lambda i: (0, 0)),
            pl.BlockSpec((1, 1), lambda i: (0, 0)),
            pl.BlockSpec((1, 1), lambda i: (0, 0)),
        ],
        out_specs=pl.BlockSpec((2, 1), lambda i: (0, 0)),
        out_shape=jax.ShapeDtypeStruct((2, 1), jnp.float32),
    )(h, wa1, ba1, wa2, ba2, cn)


def kernel(r, xyz, a, N, embed, W_in, Wf1, bf1, Wf2, bf2, Wo1, bo1, Wo2, bo2,
           Wa1, ba1, Wa2, ba2):
    a = a.astype(jnp.int32)
    dst = a[:, 0]
    src = a[:, 1]
    x = xyz[:, 0]
    y = xyz[:, 1]
    z = xyz[:, 2]

    s2 = _sc_dist(x, y, z, src, dst)
    s2 = s2.reshape(E, 1)

    wf1p = jnp.pad(Wf1, ((0, 0), (0, 128 - N_GAUSS), (0, 0)))
    w_all = _tc_filter(s2, wf1p, bf1.reshape(T, 1, 128), Wf2,
                       bf2.reshape(T, 1, 128))

    emb_pad = jnp.pad(embed, ((0, 28), (0, 0)))
    r2 = r.astype(jnp.int32)
    h, xn = _tc_embed(r2, emb_pad, W_in[0])

    zeros = jnp.zeros((NA, N_FILTERS), jnp.float32)
    for t in range(T):
        parts = _sc_msg(t, xn, w_all, src, dst, zeros)
        w_in_next = W_in[t + 1] if t + 1 < T else W_in[0]
        h, xn = _tc_update(parts, h, Wo1[t], bo1[t].reshape(1, 128),
                           Wo2[t], bo2[t].reshape(1, 128), w_in_next)

    cn = N[0].astype(jnp.int32).reshape(1, 1)
    out = _tc_head(h, Wa1, ba1.reshape(1, 64), Wa2, ba2.reshape(1, 1), cn)
    nvec = N.astype(jnp.float32).reshape(2, 1)
    return out + nvec * ba2[0]
